# Initial kernel scaffold; baseline (speedup 1.0000x reference)
import jax
import jax.numpy as jnp
from jax.experimental import pallas as pl


def kernel(pairs):
    raise NotImplementedError("write your pallas kernel here")


# speedup vs baseline: 70.5236x; 70.5236x over previous
import functools

import jax
import jax.numpy as jnp
from jax import lax
from jax.experimental import pallas as pl
from jax.experimental.pallas import tpu as pltpu
from jax.experimental.pallas import tpu_sc as plsc

_K = 5
_R = 1024
_NW = 32
_CPW = _R // _NW
_L = 16
_NVPW = _CPW // _L


@functools.lru_cache(maxsize=None)
def _sc_call(n):
    mesh = plsc.VectorSubcoreMesh(core_axis_name="c", subcore_axis_name="s")

    @functools.partial(
        pl.kernel,
        mesh=mesh,
        out_type=jax.ShapeDtypeStruct((_NW, _K, _NVPW, _L), jnp.float32),
        scratch_types=[
            pltpu.VMEM((n,), jnp.float32),
            pltpu.VMEM((n,), jnp.float32),
            pltpu.VMEM((_K, _NVPW, _L), jnp.float32),
        ],
    )
    def body(birth_hbm, death_hbm, out_hbm, b_v, d_v, o_v):
        wid = lax.axis_index("s") * 2 + lax.axis_index("c")
        pltpu.sync_copy(birth_hbm, b_v)
        pltpu.sync_copy(death_hbm, d_v)

        def red(i, carry):
            mn, mx = carry
            return (jnp.minimum(mn, b_v[pl.ds(i * _L, _L)]),
                    jnp.maximum(mx, d_v[pl.ds(i * _L, _L)]))

        mn0 = jnp.full((_L,), jnp.inf, jnp.float32)
        mx0 = jnp.full((_L,), -jnp.inf, jnp.float32)
        mn, mx = lax.fori_loop(0, n // _L, red, (mn0, mx0))
        minb = mn[0]
        maxd = mx[0]
        for i in range(1, _L):
            minb = jnp.minimum(minb, mn[i])
            maxd = jnp.maximum(maxd, mx[i])
        step = (maxd - minb) * jnp.float32(1.0 / (_R - 1))

        lane = lax.iota(jnp.int32, _L).astype(jnp.float32)
        base = (wid * _CPW).astype(jnp.float32)
        ts = tuple(minb + (base + jnp.float32(h * _L) + lane) * step
                   for h in range(_NVPW))

        init = (jnp.zeros((_L,), jnp.float32),) * (_K * _NVPW)

        def chunk_body(c, m):
            bv = b_v[pl.ds(c * _L, _L)]
            dv = d_v[pl.ds(c * _L, _L)]
            m = list(m)
            for p in range(_L):
                b = bv[p]
                d = dv[p]
                v = [jnp.minimum(jnp.maximum(t - b, 0.0),
                                 jnp.maximum(d - t, 0.0)) for t in ts]
                for i in range(_K):
                    for h in range(_NVPW):
                        mi = m[i * _NVPW + h]
                        m[i * _NVPW + h] = jnp.maximum(mi, v[h])
                        v[h] = jnp.minimum(mi, v[h])
            return tuple(m)

        m = lax.fori_loop(0, n // _L, chunk_body, init)

        for i in range(_K):
            for h in range(_NVPW):
                o_v[i, h] = m[i * _NVPW + h]
        pltpu.sync_copy(o_v, out_hbm.at[wid])

    return body


def kernel(pairs):
    n = pairs.shape[0]
    birth = pairs[:, 0]
    death = pairs[:, 1]
    out = _sc_call(n)(birth, death)
    return out.transpose(1, 0, 2, 3).reshape(_K, _R)


# speedup vs baseline: 78.6268x; 1.1149x over previous
import functools

import jax
import jax.numpy as jnp
from jax import lax
from jax.experimental import pallas as pl
from jax.experimental.pallas import tpu as pltpu
from jax.experimental.pallas import tpu_sc as plsc

_K = 5
_R = 1024
_NW = 32
_CPW = _R // _NW
_L = 16
_NVPW = _CPW // _L


@functools.lru_cache(maxsize=None)
def _sc_call(n):
    mesh = plsc.VectorSubcoreMesh(core_axis_name="c", subcore_axis_name="s")

    @functools.partial(
        pl.kernel,
        mesh=mesh,
        out_type=jax.ShapeDtypeStruct((_NW, _K, _NVPW, _L), jnp.float32),
        scratch_types=[
            pltpu.VMEM((n,), jnp.float32),
            pltpu.VMEM((n,), jnp.float32),
            pltpu.VMEM((_K, _NVPW, _L), jnp.float32),
        ],
    )
    def body(birth_hbm, death_hbm, out_hbm, b_v, d_v, o_v):
        wid = lax.axis_index("s") * 2 + lax.axis_index("c")
        pltpu.sync_copy(birth_hbm, b_v)
        pltpu.sync_copy(death_hbm, d_v)

        def red(i, carry):
            mn, mx = carry
            return (jnp.minimum(mn, b_v[pl.ds(i * _L, _L)]),
                    jnp.maximum(mx, d_v[pl.ds(i * _L, _L)]))

        mn0 = jnp.full((_L,), jnp.inf, jnp.float32)
        mx0 = jnp.full((_L,), -jnp.inf, jnp.float32)
        mn, mx = lax.fori_loop(0, n // _L, red, (mn0, mx0))
        minb = mn[0]
        maxd = mx[0]
        for i in range(1, _L):
            minb = jnp.minimum(minb, mn[i])
            maxd = jnp.maximum(maxd, mx[i])
        step = (maxd - minb) * jnp.float32(1.0 / (_R - 1))

        lane = lax.iota(jnp.int32, _L).astype(jnp.float32)
        base = (wid * _CPW).astype(jnp.float32)
        ts = tuple(minb + (base + jnp.float32(h * _L) + lane) * step
                   for h in range(_NVPW))

        init = (jnp.zeros((_L,), jnp.float32),) * (_K * _NVPW)

        def chunk_body(c, m):
            bv = b_v[pl.ds(c * _L, _L)]
            dv = d_v[pl.ds(c * _L, _L)]
            m = list(m)
            for p in range(_L):
                b = bv[p]
                d = dv[p]
                v = [jnp.maximum(jnp.minimum(t - b, d - t), 0.0) for t in ts]
                for i in range(_K):
                    for h in range(_NVPW):
                        mi = m[i * _NVPW + h]
                        m[i * _NVPW + h] = jnp.maximum(mi, v[h])
                        v[h] = jnp.minimum(mi, v[h])
            return tuple(m)

        m = lax.fori_loop(0, n // _L, chunk_body, init)

        for i in range(_K):
            for h in range(_NVPW):
                o_v[i, h] = m[i * _NVPW + h]
        pltpu.sync_copy(o_v, out_hbm.at[wid])

    return body


def kernel(pairs):
    n = pairs.shape[0]
    birth = pairs[:, 0]
    death = pairs[:, 1]
    out = _sc_call(n)(birth, death)
    return out.transpose(1, 0, 2, 3).reshape(_K, _R)


# speedup vs baseline: 111.3967x; 1.4168x over previous
import functools

import jax
import jax.numpy as jnp
from jax import lax
from jax.experimental import pallas as pl
from jax.experimental.pallas import tpu as pltpu
from jax.experimental.pallas import tpu_sc as plsc

_K = 5
_R = 1024
_INV_STEP = 1.0 / (_R - 1)

_R_SC = 512
_NW = 32
_CPW = _R_SC // _NW
_L = 16
_NVPW = _CPW // _L

_R_TC = _R - _R_SC
_W_TC = _R_TC // 8
_P_TC = 2048
_U_TC = 8


@functools.lru_cache(maxsize=None)
def _sc_call(n):
    mesh = plsc.VectorSubcoreMesh(core_axis_name="c", subcore_axis_name="s")

    @functools.partial(
        pl.kernel,
        mesh=mesh,
        out_type=jax.ShapeDtypeStruct((_NW, _K, _NVPW, _L), jnp.float32),
        scratch_types=[
            pltpu.VMEM((n,), jnp.float32),
            pltpu.VMEM((n,), jnp.float32),
            pltpu.VMEM((_K, _NVPW, _L), jnp.float32),
        ],
    )
    def body(birth_hbm, death_hbm, out_hbm, b_v, d_v, o_v):
        wid = lax.axis_index("s") * 2 + lax.axis_index("c")
        pltpu.sync_copy(birth_hbm, b_v)
        pltpu.sync_copy(death_hbm, d_v)

        def red(i, carry):
            mn, mx = carry
            return (jnp.minimum(mn, b_v[pl.ds(i * _L, _L)]),
                    jnp.maximum(mx, d_v[pl.ds(i * _L, _L)]))

        mn0 = jnp.full((_L,), jnp.inf, jnp.float32)
        mx0 = jnp.full((_L,), -jnp.inf, jnp.float32)
        mn, mx = lax.fori_loop(0, n // _L, red, (mn0, mx0))
        minb = mn[0]
        maxd = mx[0]
        for i in range(1, _L):
            minb = jnp.minimum(minb, mn[i])
            maxd = jnp.maximum(maxd, mx[i])
        step = (maxd - minb) * jnp.float32(_INV_STEP)

        lane = lax.iota(jnp.int32, _L).astype(jnp.float32)
        base = (wid * _CPW).astype(jnp.float32)
        ts = tuple(minb + (base + jnp.float32(h * _L) + lane) * step
                   for h in range(_NVPW))

        init = (jnp.zeros((_L,), jnp.float32),) * (_K * _NVPW)

        def chunk_body(c, m):
            bv = b_v[pl.ds(c * _L, _L)]
            dv = d_v[pl.ds(c * _L, _L)]
            m = list(m)
            for p in range(_L):
                b = bv[p]
                d = dv[p]
                v = [jnp.maximum(jnp.minimum(t - b, d - t), 0.0) for t in ts]
                for i in range(_K):
                    for h in range(_NVPW):
                        mi = m[i * _NVPW + h]
                        m[i * _NVPW + h] = jnp.maximum(mi, v[h])
                        v[h] = jnp.minimum(mi, v[h])
            return tuple(m)

        m = lax.fori_loop(0, n // _L, chunk_body, init)

        for i in range(_K):
            for h in range(_NVPW):
                o_v[i, h] = m[i * _NVPW + h]
        pltpu.sync_copy(o_v, out_hbm.at[wid])

    return body


def _tc_minmax_body(b_ref, d_ref, mn_ref, mx_ref):
    mn_ref[0, 0] = jnp.min(b_ref[...])
    mx_ref[0, 0] = jnp.max(d_ref[...])


def _tc_topk_body(mn_ref, mx_ref, b_ref, d_ref, o_ref):
    pid = pl.program_id(0)
    minb = mn_ref[0, 0]
    maxd = mx_ref[0, 0]
    step = (maxd - minb) * jnp.float32(_INV_STEP)
    col = (lax.broadcasted_iota(jnp.int32, (8, _W_TC), 0) * _W_TC
           + lax.broadcasted_iota(jnp.int32, (8, _W_TC), 1)
           + _R_SC).astype(jnp.float32)
    t = minb + col * step

    @pl.when(pid == 0)
    def _():
        o_ref[...] = jnp.zeros((_K, 8, _W_TC), jnp.float32)

    m = [o_ref[i] for i in range(_K)]

    def pair_body(c, m):
        m = list(m)
        for u in range(_U_TC):
            j = c * _U_TC + u
            b = b_ref[0, j]
            d = d_ref[0, j]
            v = jnp.maximum(jnp.minimum(t - b, d - t), 0.0)
            for i in range(_K):
                mi = m[i]
                m[i] = jnp.maximum(mi, v)
                v = jnp.minimum(mi, v)
        return tuple(m)

    m = lax.fori_loop(0, _P_TC // _U_TC, pair_body, tuple(m))
    for i in range(_K):
        o_ref[i] = m[i]


def _tc_call(n):
    minmax = pl.pallas_call(
        _tc_minmax_body,
        out_shape=[jax.ShapeDtypeStruct((1, 1), jnp.float32)] * 2,
        out_specs=[pl.BlockSpec(memory_space=pltpu.SMEM)] * 2,
    )
    nsteps = n // _P_TC
    topk = pl.pallas_call(
        _tc_topk_body,
        grid=(nsteps,),
        in_specs=[
            pl.BlockSpec(memory_space=pltpu.SMEM),
            pl.BlockSpec(memory_space=pltpu.SMEM),
            pl.BlockSpec((1, _P_TC), lambda i: (0, i), memory_space=pltpu.SMEM),
            pl.BlockSpec((1, _P_TC), lambda i: (0, i), memory_space=pltpu.SMEM),
        ],
        out_specs=pl.BlockSpec((_K, 8, _W_TC), lambda i: (0, 0, 0)),
        out_shape=jax.ShapeDtypeStruct((_K, 8, _W_TC), jnp.float32),
    )

    def run(birth, death):
        mn, mx = minmax(birth.reshape(8, -1), death.reshape(8, -1))
        return topk(mn, mx, birth.reshape(1, -1), death.reshape(1, -1))

    return run


def kernel(pairs):
    n = ((pairs.shape[0] + _P_TC - 1) // _P_TC) * _P_TC
    npad = n - pairs.shape[0]
    birth = jnp.pad(pairs[:, 0], (0, npad), constant_values=jnp.inf)
    death = jnp.pad(pairs[:, 1], (0, npad), constant_values=-jnp.inf)
    out_sc = _sc_call(n)(birth, death)
    out_tc = _tc_call(n)(birth, death)
    left = out_sc.transpose(1, 0, 2, 3).reshape(_K, _R_SC)
    right = out_tc.reshape(_K, _R_TC)
    return jnp.concatenate([left, right], axis=1)


# speedup vs baseline: 113.0282x; 1.0146x over previous
import functools

import jax
import jax.numpy as jnp
from jax import lax
from jax.experimental import pallas as pl
from jax.experimental.pallas import tpu as pltpu
from jax.experimental.pallas import tpu_sc as plsc

_K = 5
_R = 1024
_INV_STEP = 1.0 / (_R - 1)

_R_SC = 512
_NW = 32
_CPW = _R_SC // _NW
_L = 16
_NVPW = _CPW // _L

_R_TC = _R - _R_SC
_NB_TC = _R_TC // 128
_PAD = 1024


@functools.lru_cache(maxsize=None)
def _sc_call(n):
    mesh = plsc.VectorSubcoreMesh(core_axis_name="c", subcore_axis_name="s")

    @functools.partial(
        pl.kernel,
        mesh=mesh,
        out_type=jax.ShapeDtypeStruct((_NW, _K, _NVPW, _L), jnp.float32),
        scratch_types=[
            pltpu.VMEM((n,), jnp.float32),
            pltpu.VMEM((n,), jnp.float32),
            pltpu.VMEM((_K, _NVPW, _L), jnp.float32),
        ],
    )
    def body(birth_hbm, death_hbm, out_hbm, b_v, d_v, o_v):
        wid = lax.axis_index("s") * 2 + lax.axis_index("c")
        pltpu.sync_copy(birth_hbm, b_v)
        pltpu.sync_copy(death_hbm, d_v)

        def red(i, carry):
            mn, mx = carry
            return (jnp.minimum(mn, b_v[pl.ds(i * _L, _L)]),
                    jnp.maximum(mx, d_v[pl.ds(i * _L, _L)]))

        mn0 = jnp.full((_L,), jnp.inf, jnp.float32)
        mx0 = jnp.full((_L,), -jnp.inf, jnp.float32)
        mn, mx = lax.fori_loop(0, n // _L, red, (mn0, mx0))
        minb = mn[0]
        maxd = mx[0]
        for i in range(1, _L):
            minb = jnp.minimum(minb, mn[i])
            maxd = jnp.maximum(maxd, mx[i])
        step = (maxd - minb) * jnp.float32(_INV_STEP)

        lane = lax.iota(jnp.int32, _L).astype(jnp.float32)
        base = (wid * _CPW).astype(jnp.float32)
        ts = tuple(minb + (base + jnp.float32(h * _L) + lane) * step
                   for h in range(_NVPW))

        init = (jnp.zeros((_L,), jnp.float32),) * (_K * _NVPW)

        def chunk_body(c, m):
            bv = b_v[pl.ds(c * _L, _L)]
            dv = d_v[pl.ds(c * _L, _L)]
            m = list(m)
            for p in range(_L):
                b = bv[p]
                d = dv[p]
                v = [jnp.maximum(jnp.minimum(t - b, d - t), 0.0) for t in ts]
                for i in range(_K):
                    for h in range(_NVPW):
                        mi = m[i * _NVPW + h]
                        m[i * _NVPW + h] = jnp.maximum(mi, v[h])
                        v[h] = jnp.minimum(mi, v[h])
            return tuple(m)

        m = lax.fori_loop(0, n // _L, chunk_body, init)

        for i in range(_K):
            for h in range(_NVPW):
                o_v[i, h] = m[i * _NVPW + h]
        pltpu.sync_copy(o_v, out_hbm.at[wid])

    return body


def _tc_minmax_body(b_ref, d_ref, mn_ref, mx_ref):
    mn_ref[0, 0] = jnp.min(b_ref[...])
    mx_ref[0, 0] = jnp.max(d_ref[...])


def _tc_topk_body(mn_ref, mx_ref, b_ref, d_ref, o_ref):
    minb = mn_ref[0, 0]
    maxd = mx_ref[0, 0]
    step = (maxd - minb) * jnp.float32(_INV_STEP)
    lanef = lax.broadcasted_iota(jnp.int32, (8, 128), 1).astype(jnp.float32)
    ts = [minb + (jnp.float32(_R_SC + blk * 128) + lanef) * step
          for blk in range(_NB_TC)]

    ntile = b_ref.shape[1] // 128

    def tile_body(g, m):
        off = pl.multiple_of(g * 128, 128)
        bt = b_ref[:, pl.ds(off, 128)]
        dt = d_ref[:, pl.ds(off, 128)]
        m = list(m)
        for u in range(128):
            b8 = lax.slice(bt, (0, u), (8, u + 1))
            d8 = lax.slice(dt, (0, u), (8, u + 1))
            for blk in range(_NB_TC):
                v = jnp.maximum(jnp.minimum(ts[blk] - b8, d8 - ts[blk]), 0.0)
                for i in range(_K):
                    mi = m[blk * _K + i]
                    m[blk * _K + i] = jnp.maximum(mi, v)
                    v = jnp.minimum(mi, v)
        return tuple(m)

    init = (jnp.zeros((8, 128), jnp.float32),) * (_NB_TC * _K)
    m = lax.fori_loop(0, ntile, tile_body, init)

    for blk in range(_NB_TC):
        fin = [jnp.zeros((1, 128), jnp.float32) for _ in range(_K)]
        for s in range(8):
            for i in range(_K):
                v = lax.slice(m[blk * _K + i], (s, 0), (s + 1, 128))
                for q in range(i, _K):
                    fq = fin[q]
                    fin[q] = jnp.maximum(fq, v)
                    v = jnp.minimum(fq, v)
        for i in range(_K):
            o_ref[pl.ds(i, 1), pl.ds(blk * 128, 128)] = fin[i]


def _tc_call(n):
    minmax = pl.pallas_call(
        _tc_minmax_body,
        out_shape=[jax.ShapeDtypeStruct((1, 1), jnp.float32)] * 2,
        out_specs=[pl.BlockSpec(memory_space=pltpu.SMEM)] * 2,
    )
    topk = pl.pallas_call(
        _tc_topk_body,
        in_specs=[
            pl.BlockSpec(memory_space=pltpu.SMEM),
            pl.BlockSpec(memory_space=pltpu.SMEM),
            pl.BlockSpec(memory_space=pltpu.VMEM),
            pl.BlockSpec(memory_space=pltpu.VMEM),
        ],
        out_shape=jax.ShapeDtypeStruct((_K, _R_TC), jnp.float32),
    )

    def run(birth, death):
        mn, mx = minmax(birth.reshape(8, -1), death.reshape(8, -1))
        bt = birth.reshape(-1, 8).T
        dt = death.reshape(-1, 8).T
        return topk(mn, mx, bt, dt)

    return run


def kernel(pairs):
    n = ((pairs.shape[0] + _PAD - 1) // _PAD) * _PAD
    npad = n - pairs.shape[0]
    birth = jnp.pad(pairs[:, 0], (0, npad), constant_values=jnp.inf)
    death = jnp.pad(pairs[:, 1], (0, npad), constant_values=-jnp.inf)
    out_sc = _sc_call(n)(birth, death)
    out_tc = _tc_call(n)(birth, death)
    left = out_sc.transpose(1, 0, 2, 3).reshape(_K, _R_SC)
    right = out_tc.reshape(_K, _R_TC)
    return jnp.concatenate([left, right], axis=1)


# speedup vs baseline: 159.6519x; 1.4125x over previous
import functools

import jax
import jax.numpy as jnp
from jax import lax
from jax.experimental import pallas as pl
from jax.experimental.pallas import tpu as pltpu
from jax.experimental.pallas import tpu_sc as plsc

_K = 5
_R = 1024
_INV_STEP = 1.0 / (_R - 1)

_R_SC = 256
_NW = 32
_G = 2
_NSETS = _NW // _G
_CPW = _R_SC // _NSETS
_L = 16
_NVPW = _CPW // _L

_R_TC = _R - _R_SC
_NB_TC = _R_TC // 128
_PAD = 1024


@functools.lru_cache(maxsize=None)
def _sc_call(n):
    mesh = plsc.VectorSubcoreMesh(core_axis_name="c", subcore_axis_name="s")

    @functools.partial(
        pl.kernel,
        mesh=mesh,
        out_type=jax.ShapeDtypeStruct((_NW, _K, _NVPW, _L), jnp.float32),
        scratch_types=[
            pltpu.VMEM((n,), jnp.float32),
            pltpu.VMEM((n,), jnp.float32),
            pltpu.VMEM((_K, _NVPW, _L), jnp.float32),
        ],
    )
    def body(birth_hbm, death_hbm, out_hbm, b_v, d_v, o_v):
        wid = lax.axis_index("s") * 2 + lax.axis_index("c")
        grp = wid & (_G - 1)
        cset = wid >> 1
        pltpu.sync_copy(birth_hbm, b_v)
        pltpu.sync_copy(death_hbm, d_v)

        def red(i, carry):
            mn, mx = carry
            return (jnp.minimum(mn, b_v[pl.ds(i * _L, _L)]),
                    jnp.maximum(mx, d_v[pl.ds(i * _L, _L)]))

        mn0 = jnp.full((_L,), jnp.inf, jnp.float32)
        mx0 = jnp.full((_L,), -jnp.inf, jnp.float32)
        mn, mx = lax.fori_loop(0, n // _L, red, (mn0, mx0))
        minb = mn[0]
        maxd = mx[0]
        for i in range(1, _L):
            minb = jnp.minimum(minb, mn[i])
            maxd = jnp.maximum(maxd, mx[i])
        step = (maxd - minb) * jnp.float32(_INV_STEP)

        lane = lax.iota(jnp.int32, _L).astype(jnp.float32)
        base = (cset * _CPW).astype(jnp.float32)
        ts = tuple(minb + (base + jnp.float32(h * _L) + lane) * step
                   for h in range(_NVPW))

        init = (jnp.zeros((_L,), jnp.float32),) * (_K * _NVPW)

        def chunk_body(c, m):
            bv = b_v[pl.ds(c * _L, _L)]
            dv = d_v[pl.ds(c * _L, _L)]
            m = list(m)
            for p in range(_L):
                b = bv[p]
                d = dv[p]
                v = [jnp.maximum(jnp.minimum(t - b, d - t), 0.0) for t in ts]
                for i in range(_K):
                    for h in range(_NVPW):
                        mi = m[i * _NVPW + h]
                        m[i * _NVPW + h] = jnp.maximum(mi, v[h])
                        v[h] = jnp.minimum(mi, v[h])
            return tuple(m)

        nch_half = n // _L // _G
        m = lax.fori_loop(grp * nch_half, (grp + 1) * nch_half,
                          chunk_body, init)

        for i in range(_K):
            for h in range(_NVPW):
                o_v[i, h] = m[i * _NVPW + h]
        pltpu.sync_copy(o_v, out_hbm.at[grp * _NSETS + cset])

    return body


def _tc_minmax_body(b_ref, d_ref, mn_ref, mx_ref):
    mn_ref[0, 0] = jnp.min(b_ref[...])
    mx_ref[0, 0] = jnp.max(d_ref[...])


def _tc_topk_body(mn_ref, mx_ref, b_ref, d_ref, o_ref):
    minb = mn_ref[0, 0]
    maxd = mx_ref[0, 0]
    step = (maxd - minb) * jnp.float32(_INV_STEP)
    lanef = lax.broadcasted_iota(jnp.int32, (8, 128), 1).astype(jnp.float32)
    ts = [minb + (jnp.float32(_R_SC + blk * 128) + lanef) * step
          for blk in range(_NB_TC)]

    ntile = b_ref.shape[1] // 128

    def tile_body(g, m):
        off = pl.multiple_of(g * 128, 128)
        bt = b_ref[:, pl.ds(off, 128)]
        dt = d_ref[:, pl.ds(off, 128)]
        m = list(m)
        for u in range(128):
            b8 = lax.slice(bt, (0, u), (8, u + 1))
            d8 = lax.slice(dt, (0, u), (8, u + 1))
            for blk in range(_NB_TC):
                v = jnp.maximum(jnp.minimum(ts[blk] - b8, d8 - ts[blk]), 0.0)
                for i in range(_K):
                    mi = m[blk * _K + i]
                    m[blk * _K + i] = jnp.maximum(mi, v)
                    v = jnp.minimum(mi, v)
        return tuple(m)

    init = (jnp.zeros((8, 128), jnp.float32),) * (_NB_TC * _K)
    m = lax.fori_loop(0, ntile, tile_body, init)

    for blk in range(_NB_TC):
        fin = [jnp.zeros((1, 128), jnp.float32) for _ in range(_K)]
        for s in range(8):
            for i in range(_K):
                v = lax.slice(m[blk * _K + i], (s, 0), (s + 1, 128))
                for q in range(i, _K):
                    fq = fin[q]
                    fin[q] = jnp.maximum(fq, v)
                    v = jnp.minimum(fq, v)
        for i in range(_K):
            o_ref[pl.ds(i, 1), pl.ds(blk * 128, 128)] = fin[i]


def _tc_merge_body(a_ref, b_ref, o_ref):
    fin = [a_ref[pl.ds(i, 1), :] for i in range(_K)]
    for i in range(_K):
        v = b_ref[pl.ds(i, 1), :]
        for q in range(i, _K):
            fq = fin[q]
            fin[q] = jnp.maximum(fq, v)
            v = jnp.minimum(fq, v)
    for i in range(_K):
        o_ref[pl.ds(i, 1), :] = fin[i]


def _tc_call(n):
    minmax = pl.pallas_call(
        _tc_minmax_body,
        out_shape=[jax.ShapeDtypeStruct((1, 1), jnp.float32)] * 2,
        out_specs=[pl.BlockSpec(memory_space=pltpu.SMEM)] * 2,
    )
    topk = pl.pallas_call(
        _tc_topk_body,
        in_specs=[
            pl.BlockSpec(memory_space=pltpu.SMEM),
            pl.BlockSpec(memory_space=pltpu.SMEM),
            pl.BlockSpec(memory_space=pltpu.VMEM),
            pl.BlockSpec(memory_space=pltpu.VMEM),
        ],
        out_shape=jax.ShapeDtypeStruct((_K, _R_TC), jnp.float32),
    )

    merge = pl.pallas_call(
        _tc_merge_body,
        out_shape=jax.ShapeDtypeStruct((_K, _R_SC), jnp.float32),
    )

    def run(birth, death):
        mn, mx = minmax(birth.reshape(8, -1), death.reshape(8, -1))
        bt = birth.reshape(-1, 8).T
        dt = death.reshape(-1, 8).T
        return topk(mn, mx, bt, dt), merge

    return run


def kernel(pairs):
    n = ((pairs.shape[0] + _PAD - 1) // _PAD) * _PAD
    npad = n - pairs.shape[0]
    birth = jnp.pad(pairs[:, 0], (0, npad), constant_values=jnp.inf)
    death = jnp.pad(pairs[:, 1], (0, npad), constant_values=-jnp.inf)
    out_sc = _sc_call(n)(birth, death)
    out_tc, merge = _tc_call(n)(birth, death)
    halves = out_sc.reshape(_G, _NSETS, _K, _NVPW, _L)
    a = halves[0].transpose(1, 0, 2, 3).reshape(_K, _R_SC)
    b = halves[1].transpose(1, 0, 2, 3).reshape(_K, _R_SC)
    left = merge(a, b)
    return jnp.concatenate([left, out_tc], axis=1)


# speedup vs baseline: 164.3492x; 1.0294x over previous
import functools

import jax
import jax.numpy as jnp
from jax import lax
from jax.experimental import pallas as pl
from jax.experimental.pallas import tpu as pltpu
from jax.experimental.pallas import tpu_sc as plsc

_K = 5
_R = 1024
_INV_STEP = 1.0 / (_R - 1)

_R_SC = 128
_NW = 32
_G = 4
_NSETS = _NW // _G
_CPW = _R_SC // _NSETS
_L = 16
_NVPW = _CPW // _L

_R_TC = _R - _R_SC
_NB_TC = _R_TC // 128
_PAD = 1024


@functools.lru_cache(maxsize=None)
def _sc_call(n):
    mesh = plsc.VectorSubcoreMesh(core_axis_name="c", subcore_axis_name="s")

    @functools.partial(
        pl.kernel,
        mesh=mesh,
        out_type=jax.ShapeDtypeStruct((_NW, _K, _NVPW, _L), jnp.float32),
        scratch_types=[
            pltpu.VMEM((n,), jnp.float32),
            pltpu.VMEM((n,), jnp.float32),
            pltpu.VMEM((_K, _NVPW, _L), jnp.float32),
        ],
    )
    def body(birth_hbm, death_hbm, out_hbm, b_v, d_v, o_v):
        wid = lax.axis_index("s") * 2 + lax.axis_index("c")
        grp = wid & (_G - 1)
        cset = wid >> 2
        pltpu.sync_copy(birth_hbm, b_v)
        pltpu.sync_copy(death_hbm, d_v)

        def red(i, carry):
            mn, mx = carry
            return (jnp.minimum(mn, b_v[pl.ds(i * _L, _L)]),
                    jnp.maximum(mx, d_v[pl.ds(i * _L, _L)]))

        mn0 = jnp.full((_L,), jnp.inf, jnp.float32)
        mx0 = jnp.full((_L,), -jnp.inf, jnp.float32)
        mn, mx = lax.fori_loop(0, n // _L, red, (mn0, mx0))
        minb = mn[0]
        maxd = mx[0]
        for i in range(1, _L):
            minb = jnp.minimum(minb, mn[i])
            maxd = jnp.maximum(maxd, mx[i])
        step = (maxd - minb) * jnp.float32(_INV_STEP)

        lane = lax.iota(jnp.int32, _L).astype(jnp.float32)
        base = (cset * _CPW).astype(jnp.float32)
        ts = tuple(minb + (base + jnp.float32(h * _L) + lane) * step
                   for h in range(_NVPW))

        init = (jnp.zeros((_L,), jnp.float32),) * (_K * _NVPW)

        def chunk_body(c, m):
            bv = b_v[pl.ds(c * _L, _L)]
            dv = d_v[pl.ds(c * _L, _L)]
            m = list(m)
            for p in range(_L):
                b = bv[p]
                d = dv[p]
                v = [jnp.maximum(jnp.minimum(t - b, d - t), 0.0) for t in ts]
                for i in range(_K):
                    for h in range(_NVPW):
                        mi = m[i * _NVPW + h]
                        m[i * _NVPW + h] = jnp.maximum(mi, v[h])
                        v[h] = jnp.minimum(mi, v[h])
            return tuple(m)

        nch_half = n // _L // _G
        m = lax.fori_loop(grp * nch_half, (grp + 1) * nch_half,
                          chunk_body, init)

        for i in range(_K):
            for h in range(_NVPW):
                o_v[i, h] = m[i * _NVPW + h]
        pltpu.sync_copy(o_v, out_hbm.at[grp * _NSETS + cset])

    return body


def _tc_topk_body(b_ref, d_ref, o_ref):
    minb = jnp.min(b_ref[...])
    maxd = jnp.max(d_ref[...])
    step = (maxd - minb) * jnp.float32(_INV_STEP)
    lanef = lax.broadcasted_iota(jnp.int32, (8, 128), 1).astype(jnp.float32)
    ts = [minb + (jnp.float32(_R_SC + blk * 128) + lanef) * step
          for blk in range(_NB_TC)]

    ntile = b_ref.shape[1] // 128

    def tile_body(g, m):
        off = pl.multiple_of(g * 128, 128)
        bt = b_ref[:, pl.ds(off, 128)]
        dt = d_ref[:, pl.ds(off, 128)]
        m = list(m)
        for u in range(128):
            b8 = lax.slice(bt, (0, u), (8, u + 1))
            d8 = lax.slice(dt, (0, u), (8, u + 1))
            for blk in range(_NB_TC):
                v = jnp.maximum(jnp.minimum(ts[blk] - b8, d8 - ts[blk]), 0.0)
                for i in range(_K):
                    mi = m[blk * _K + i]
                    m[blk * _K + i] = jnp.maximum(mi, v)
                    v = jnp.minimum(mi, v)
        return tuple(m)

    init = (jnp.zeros((8, 128), jnp.float32),) * (_NB_TC * _K)
    m = lax.fori_loop(0, ntile, tile_body, init)

    for blk in range(_NB_TC):
        fin = [jnp.zeros((1, 128), jnp.float32) for _ in range(_K)]
        for s in range(8):
            for i in range(_K):
                v = lax.slice(m[blk * _K + i], (s, 0), (s + 1, 128))
                for q in range(i, _K):
                    fq = fin[q]
                    fin[q] = jnp.maximum(fq, v)
                    v = jnp.minimum(fq, v)
        for i in range(_K):
            o_ref[pl.ds(i, 1), pl.ds(blk * 128, 128)] = fin[i]


def _tc_merge_body(*refs):
    parts, o_ref = refs[:-1], refs[-1]
    fin = [parts[0][pl.ds(i, 1), :] for i in range(_K)]
    for part in parts[1:]:
        for i in range(_K):
            v = part[pl.ds(i, 1), :]
            for q in range(i, _K):
                fq = fin[q]
                fin[q] = jnp.maximum(fq, v)
                v = jnp.minimum(fq, v)
    for i in range(_K):
        o_ref[pl.ds(i, 1), :] = fin[i]


def _tc_call(n):
    topk = pl.pallas_call(
        _tc_topk_body,
        out_shape=jax.ShapeDtypeStruct((_K, _R_TC), jnp.float32),
    )
    merge = pl.pallas_call(
        _tc_merge_body,
        out_shape=jax.ShapeDtypeStruct((_K, _R_SC), jnp.float32),
    )

    def run(birth, death):
        bt = birth.reshape(-1, 8).T
        dt = death.reshape(-1, 8).T
        return topk(bt, dt), merge

    return run


def kernel(pairs):
    n = ((pairs.shape[0] + _PAD - 1) // _PAD) * _PAD
    npad = n - pairs.shape[0]
    birth = jnp.pad(pairs[:, 0], (0, npad), constant_values=jnp.inf)
    death = jnp.pad(pairs[:, 1], (0, npad), constant_values=-jnp.inf)
    out_sc = _sc_call(n)(birth, death)
    out_tc, merge = _tc_call(n)(birth, death)
    parts = out_sc.reshape(_G, _NSETS, _K, _NVPW, _L)
    parts = [parts[g].transpose(1, 0, 2, 3).reshape(_K, _R_SC)
             for g in range(_G)]
    left = merge(*parts)
    return jnp.concatenate([left, out_tc], axis=1)


# speedup vs baseline: 174.8300x; 1.0638x over previous
import functools

import jax
import jax.numpy as jnp
from jax import lax
from jax.experimental import pallas as pl
from jax.experimental.pallas import tpu as pltpu
from jax.experimental.pallas import tpu_sc as plsc

_K = 5
_R = 1024
_INV_STEP = 1.0 / (_R - 1)

_NW = 32
_CPW = _R // _NW
_L = 16
_NVPW = _CPW // _L

_NB_TC = _R // 128
_PAD = 1024
_SC_TILES = 3


@functools.lru_cache(maxsize=None)
def _sc_call(n):
    mesh = plsc.VectorSubcoreMesh(core_axis_name="c", subcore_axis_name="s")
    n_sc = _SC_TILES * _PAD

    @functools.partial(
        pl.kernel,
        mesh=mesh,
        out_type=jax.ShapeDtypeStruct((_NW, _K, _NVPW, _L), jnp.float32),
        scratch_types=[
            pltpu.VMEM((n,), jnp.float32),
            pltpu.VMEM((n,), jnp.float32),
            pltpu.VMEM((_K, _NVPW, _L), jnp.float32),
        ],
    )
    def body(birth_hbm, death_hbm, out_hbm, b_v, d_v, o_v):
        wid = lax.axis_index("s") * 2 + lax.axis_index("c")
        pltpu.sync_copy(birth_hbm, b_v)
        pltpu.sync_copy(death_hbm, d_v)

        def red(i, carry):
            mn, mx = carry
            return (jnp.minimum(mn, b_v[pl.ds(i * _L, _L)]),
                    jnp.maximum(mx, d_v[pl.ds(i * _L, _L)]))

        mn0 = jnp.full((_L,), jnp.inf, jnp.float32)
        mx0 = jnp.full((_L,), -jnp.inf, jnp.float32)
        mn, mx = lax.fori_loop(0, n // _L, red, (mn0, mx0))
        minb = mn[0]
        maxd = mx[0]
        for i in range(1, _L):
            minb = jnp.minimum(minb, mn[i])
            maxd = jnp.maximum(maxd, mx[i])
        step = (maxd - minb) * jnp.float32(_INV_STEP)

        lane = lax.iota(jnp.int32, _L).astype(jnp.float32)
        base = (wid * _CPW).astype(jnp.float32)
        ts = tuple(minb + (base + jnp.float32(h * _L) + lane) * step
                   for h in range(_NVPW))

        init = (jnp.zeros((_L,), jnp.float32),) * (_K * _NVPW)

        def chunk_body(c, m):
            bv = b_v[pl.ds(c * _L, _L)]
            dv = d_v[pl.ds(c * _L, _L)]
            m = list(m)
            for p in range(_L):
                b = bv[p]
                d = dv[p]
                v = [jnp.maximum(jnp.minimum(t - b, d - t), 0.0) for t in ts]
                for i in range(_K):
                    for h in range(_NVPW):
                        mi = m[i * _NVPW + h]
                        m[i * _NVPW + h] = jnp.maximum(mi, v[h])
                        v[h] = jnp.minimum(mi, v[h])
            return tuple(m)

        m = lax.fori_loop(0, n_sc // _L, chunk_body, init)

        for i in range(_K):
            for h in range(_NVPW):
                o_v[i, h] = m[i * _NVPW + h]
        pltpu.sync_copy(o_v, out_hbm.at[wid])

    return body


def _tc_topk_body(b_ref, d_ref, o_ref):
    minb = jnp.min(b_ref[...])
    maxd = jnp.max(d_ref[...])
    step = (maxd - minb) * jnp.float32(_INV_STEP)
    lanef = lax.broadcasted_iota(jnp.int32, (8, 128), 1).astype(jnp.float32)
    ts = [minb + (jnp.float32(blk * 128) + lanef) * step
          for blk in range(_NB_TC)]

    ntile = b_ref.shape[1] // 128

    def tile_body(g, m):
        off = pl.multiple_of(g * 128, 128)
        bt = b_ref[:, pl.ds(off, 128)]
        dt = d_ref[:, pl.ds(off, 128)]
        m = list(m)
        for u in range(128):
            b8 = lax.slice(bt, (0, u), (8, u + 1))
            d8 = lax.slice(dt, (0, u), (8, u + 1))
            for blk in range(_NB_TC):
                v = jnp.maximum(jnp.minimum(ts[blk] - b8, d8 - ts[blk]), 0.0)
                for i in range(_K):
                    mi = m[blk * _K + i]
                    m[blk * _K + i] = jnp.maximum(mi, v)
                    v = jnp.minimum(mi, v)
        return tuple(m)

    init = (jnp.zeros((8, 128), jnp.float32),) * (_NB_TC * _K)
    m = lax.fori_loop(_SC_TILES * _PAD // 1024, ntile, tile_body, init)

    for blk in range(_NB_TC):
        fin = [jnp.zeros((1, 128), jnp.float32) for _ in range(_K)]
        for s in range(8):
            for i in range(_K):
                v = lax.slice(m[blk * _K + i], (s, 0), (s + 1, 128))
                for q in range(i, _K):
                    fq = fin[q]
                    fin[q] = jnp.maximum(fq, v)
                    v = jnp.minimum(fq, v)
        for i in range(_K):
            o_ref[pl.ds(i, 1), pl.ds(blk * 128, 128)] = fin[i]


def _tc_merge_body(a_ref, b_ref, o_ref):
    fin = [a_ref[pl.ds(i, 1), :] for i in range(_K)]
    for i in range(_K):
        v = b_ref[pl.ds(i, 1), :]
        for q in range(i, _K):
            fq = fin[q]
            fin[q] = jnp.maximum(fq, v)
            v = jnp.minimum(fq, v)
    for i in range(_K):
        o_ref[pl.ds(i, 1), :] = fin[i]


def _tc_call(n):
    topk = pl.pallas_call(
        _tc_topk_body,
        out_shape=jax.ShapeDtypeStruct((_K, _R), jnp.float32),
    )
    merge = pl.pallas_call(
        _tc_merge_body,
        out_shape=jax.ShapeDtypeStruct((_K, _R), jnp.float32),
    )

    def run(birth, death):
        bt = birth.reshape(-1, 8).T
        dt = death.reshape(-1, 8).T
        return topk(bt, dt), merge

    return run


def kernel(pairs):
    n = ((pairs.shape[0] + _PAD - 1) // _PAD) * _PAD
    npad = n - pairs.shape[0]
    birth = jnp.pad(pairs[:, 0], (0, npad), constant_values=jnp.inf)
    death = jnp.pad(pairs[:, 1], (0, npad), constant_values=-jnp.inf)
    out_sc = _sc_call(n)(birth, death)
    out_tc, merge = _tc_call(n)(birth, death)
    sc_flat = out_sc.transpose(1, 0, 2, 3).reshape(_K, _R)
    return merge(out_tc, sc_flat)


# speedup vs baseline: 187.5226x; 1.0726x over previous
import functools

import jax
import jax.numpy as jnp
from jax import lax
from jax.experimental import pallas as pl
from jax.experimental.pallas import tpu as pltpu
from jax.experimental.pallas import tpu_sc as plsc

_K = 5
_R = 1024
_INV_STEP = 1.0 / (_R - 1)

_NW = 32
_CPW = _R // _NW
_L = 16
_NVPW = _CPW // _L

_NB_TC = _R // 128
_PAD = 1024
_SC_TILES = 3


@functools.lru_cache(maxsize=None)
def _sc_call(n):
    mesh = plsc.VectorSubcoreMesh(core_axis_name="c", subcore_axis_name="s")
    n_sc = _SC_TILES * _PAD

    @functools.partial(
        pl.kernel,
        mesh=mesh,
        out_type=jax.ShapeDtypeStruct((_K, _R), jnp.float32),
        scratch_types=[
            pltpu.VMEM((n,), jnp.float32),
            pltpu.VMEM((n,), jnp.float32),
            pltpu.VMEM((_K, _CPW), jnp.float32),
        ],
    )
    def body(birth_hbm, death_hbm, out_hbm, b_v, d_v, o_v):
        wid = lax.axis_index("s") * 2 + lax.axis_index("c")
        pltpu.sync_copy(birth_hbm, b_v)
        pltpu.sync_copy(death_hbm, d_v)

        def red(i, carry):
            mn, mx = carry
            return (jnp.minimum(mn, b_v[pl.ds(i * _L, _L)]),
                    jnp.maximum(mx, d_v[pl.ds(i * _L, _L)]))

        mn0 = jnp.full((_L,), jnp.inf, jnp.float32)
        mx0 = jnp.full((_L,), -jnp.inf, jnp.float32)
        mn, mx = lax.fori_loop(0, n // _L, red, (mn0, mx0))
        minb = mn[0]
        maxd = mx[0]
        for i in range(1, _L):
            minb = jnp.minimum(minb, mn[i])
            maxd = jnp.maximum(maxd, mx[i])
        step = (maxd - minb) * jnp.float32(_INV_STEP)

        lane = lax.iota(jnp.int32, _L).astype(jnp.float32)
        base = (wid * _CPW).astype(jnp.float32)
        ts = tuple(minb + (base + jnp.float32(h * _L) + lane) * step
                   for h in range(_NVPW))

        init = (jnp.zeros((_L,), jnp.float32),) * (_K * _NVPW)

        def chunk_body(c, m):
            bv = b_v[pl.ds(c * _L, _L)]
            dv = d_v[pl.ds(c * _L, _L)]
            m = list(m)
            for p in range(_L):
                b = bv[p]
                d = dv[p]
                v = [jnp.maximum(jnp.minimum(t - b, d - t), 0.0) for t in ts]
                for i in range(_K):
                    for h in range(_NVPW):
                        mi = m[i * _NVPW + h]
                        m[i * _NVPW + h] = jnp.maximum(mi, v[h])
                        v[h] = jnp.minimum(mi, v[h])
            return tuple(m)

        per_sub = n // 8 // _L
        sc_sub = n_sc // 8 // _L
        m = init
        for s in range(8):
            m = lax.fori_loop(s * per_sub, s * per_sub + sc_sub,
                              chunk_body, m)

        for i in range(_K):
            for h in range(_NVPW):
                o_v[i, pl.ds(h * _L, _L)] = m[i * _NVPW + h]
        for i in range(_K):
            pltpu.sync_copy(o_v.at[i],
                            out_hbm.at[i, pl.ds(wid * _CPW, _CPW)])

    return body


def _tc_topk_body(b_ref, d_ref, o_ref):
    minb = jnp.min(b_ref[...])
    maxd = jnp.max(d_ref[...])
    step = (maxd - minb) * jnp.float32(_INV_STEP)
    lanef = lax.broadcasted_iota(jnp.int32, (8, 128), 1).astype(jnp.float32)
    ts = [minb + (jnp.float32(blk * 128) + lanef) * step
          for blk in range(_NB_TC)]

    ntile = b_ref.shape[1] // 128

    def tile_body(g, m):
        off = pl.multiple_of(g * 128, 128)
        bt = b_ref[:, pl.ds(off, 128)]
        dt = d_ref[:, pl.ds(off, 128)]
        m = list(m)
        for u in range(128):
            b8 = lax.slice(bt, (0, u), (8, u + 1))
            d8 = lax.slice(dt, (0, u), (8, u + 1))
            for blk in range(_NB_TC):
                v = jnp.maximum(jnp.minimum(ts[blk] - b8, d8 - ts[blk]), 0.0)
                for i in range(_K):
                    mi = m[blk * _K + i]
                    m[blk * _K + i] = jnp.maximum(mi, v)
                    v = jnp.minimum(mi, v)
        return tuple(m)

    init = (jnp.zeros((8, 128), jnp.float32),) * (_NB_TC * _K)
    m = lax.fori_loop(_SC_TILES * _PAD // 1024, ntile, tile_body, init)

    for blk in range(_NB_TC):
        fin = [jnp.zeros((1, 128), jnp.float32) for _ in range(_K)]
        for s in range(8):
            for i in range(_K):
                v = lax.slice(m[blk * _K + i], (s, 0), (s + 1, 128))
                for q in range(i, _K):
                    fq = fin[q]
                    fin[q] = jnp.maximum(fq, v)
                    v = jnp.minimum(fq, v)
        for i in range(_K):
            o_ref[pl.ds(i, 1), pl.ds(blk * 128, 128)] = fin[i]


def _tc_merge_body(a_ref, b_ref, o_ref):
    fin = [a_ref[pl.ds(i, 1), :] for i in range(_K)]
    for i in range(_K):
        v = b_ref[pl.ds(i, 1), :]
        for q in range(i, _K):
            fq = fin[q]
            fin[q] = jnp.maximum(fq, v)
            v = jnp.minimum(fq, v)
    for i in range(_K):
        o_ref[pl.ds(i, 1), :] = fin[i]


def _tc_call(n):
    topk = pl.pallas_call(
        _tc_topk_body,
        out_shape=jax.ShapeDtypeStruct((_K, _R), jnp.float32),
    )
    merge = pl.pallas_call(
        _tc_merge_body,
        out_shape=jax.ShapeDtypeStruct((_K, _R), jnp.float32),
    )

    def run(birth, death):
        bt = birth.reshape(8, -1)
        dt = death.reshape(8, -1)
        return topk(bt, dt), merge

    return run


def kernel(pairs):
    n = ((pairs.shape[0] + _PAD - 1) // _PAD) * _PAD
    npad = n - pairs.shape[0]
    birth = jnp.pad(pairs[:, 0], (0, npad), constant_values=jnp.inf)
    death = jnp.pad(pairs[:, 1], (0, npad), constant_values=-jnp.inf)
    out_sc = _sc_call(n)(birth, death)
    out_tc, merge = _tc_call(n)(birth, death)
    return merge(out_tc, out_sc)


# speedup vs baseline: 197.9895x; 1.0558x over previous
import functools

import jax
import jax.numpy as jnp
from jax import lax
from jax.experimental import pallas as pl
from jax.experimental.pallas import tpu as pltpu
from jax.experimental.pallas import tpu_sc as plsc

_K = 5
_R = 1024
_INV_STEP = 1.0 / (_R - 1)

_NW = 32
_CPW = _R // _NW
_L = 16
_NVPW = _CPW // _L

_NB_TC = _R // 128
_PAD = 1024
_SC_TILES = 3


@functools.lru_cache(maxsize=None)
def _sc_call(n):
    mesh = plsc.VectorSubcoreMesh(core_axis_name="c", subcore_axis_name="s")
    n_sc = _SC_TILES * _PAD

    @functools.partial(
        pl.kernel,
        mesh=mesh,
        out_type=jax.ShapeDtypeStruct((_K, _R), jnp.float32),
        scratch_types=[
            pltpu.VMEM((n,), jnp.float32),
            pltpu.VMEM((n,), jnp.float32),
            pltpu.VMEM((_K, _CPW), jnp.float32),
        ],
    )
    def body(birth_hbm, death_hbm, out_hbm, b_v, d_v, o_v):
        wid = lax.axis_index("s") * 2 + lax.axis_index("c")
        pltpu.sync_copy(birth_hbm, b_v)
        pltpu.sync_copy(death_hbm, d_v)

        def red(i, carry):
            mn, mx = carry
            return (jnp.minimum(mn, b_v[pl.ds(i * _L, _L)]),
                    jnp.maximum(mx, d_v[pl.ds(i * _L, _L)]))

        mn0 = jnp.full((_L,), jnp.inf, jnp.float32)
        mx0 = jnp.full((_L,), -jnp.inf, jnp.float32)
        mn, mx = lax.fori_loop(0, n // _L, red, (mn0, mx0))
        minb = mn[0]
        maxd = mx[0]
        for i in range(1, _L):
            minb = jnp.minimum(minb, mn[i])
            maxd = jnp.maximum(maxd, mx[i])
        step = (maxd - minb) * jnp.float32(_INV_STEP)

        lane = lax.iota(jnp.int32, _L).astype(jnp.float32)
        base = (wid * _CPW).astype(jnp.float32)
        ts = tuple(minb + (base + jnp.float32(h * _L) + lane) * step
                   for h in range(_NVPW))

        init = (jnp.zeros((_L,), jnp.float32),) * (_K * _NVPW)

        def chunk_body(c, m):
            bv = b_v[pl.ds(c * _L, _L)]
            dv = d_v[pl.ds(c * _L, _L)]
            m = list(m)
            for p in range(_L):
                b = bv[p]
                d = dv[p]
                v = [jnp.minimum(t - b, d - t) for t in ts]
                for i in range(_K):
                    for h in range(_NVPW):
                        mi = m[i * _NVPW + h]
                        m[i * _NVPW + h] = jnp.maximum(mi, v[h])
                        v[h] = jnp.minimum(mi, v[h])
            return tuple(m)

        per_sub = n // 8 // _L
        sc_sub = n_sc // 8 // _L
        m = init
        for s in range(8):
            m = lax.fori_loop(s * per_sub, s * per_sub + sc_sub,
                              chunk_body, m)

        for i in range(_K):
            for h in range(_NVPW):
                o_v[i, pl.ds(h * _L, _L)] = m[i * _NVPW + h]
        for i in range(_K):
            pltpu.sync_copy(o_v.at[i],
                            out_hbm.at[i, pl.ds(wid * _CPW, _CPW)])

    return body


def _tc_topk_body(b_ref, d_ref, o_ref):
    minb = jnp.min(b_ref[...])
    maxd = jnp.max(d_ref[...])
    step = (maxd - minb) * jnp.float32(_INV_STEP)
    lanef = lax.broadcasted_iota(jnp.int32, (8, 128), 1).astype(jnp.float32)
    ts = [minb + (jnp.float32(blk * 128) + lanef) * step
          for blk in range(_NB_TC)]

    ntile = b_ref.shape[1] // 128

    def tile_body(g, m):
        off = pl.multiple_of(g * 128, 128)
        bt = b_ref[:, pl.ds(off, 128)]
        dt = d_ref[:, pl.ds(off, 128)]
        m = list(m)
        for u in range(128):
            b8 = lax.slice(bt, (0, u), (8, u + 1))
            d8 = lax.slice(dt, (0, u), (8, u + 1))
            for blk in range(_NB_TC):
                v = jnp.minimum(ts[blk] - b8, d8 - ts[blk])
                for i in range(_K):
                    mi = m[blk * _K + i]
                    m[blk * _K + i] = jnp.maximum(mi, v)
                    v = jnp.minimum(mi, v)
        return tuple(m)

    init = (jnp.zeros((8, 128), jnp.float32),) * (_NB_TC * _K)
    m = lax.fori_loop(_SC_TILES * _PAD // 1024, ntile, tile_body, init)

    for blk in range(_NB_TC):
        fin = [jnp.zeros((1, 128), jnp.float32) for _ in range(_K)]
        for s in range(8):
            for i in range(_K):
                v = lax.slice(m[blk * _K + i], (s, 0), (s + 1, 128))
                for q in range(i, _K):
                    fq = fin[q]
                    fin[q] = jnp.maximum(fq, v)
                    v = jnp.minimum(fq, v)
        for i in range(_K):
            o_ref[pl.ds(i, 1), pl.ds(blk * 128, 128)] = fin[i]


def _tc_merge_body(a_ref, b_ref, o_ref):
    fin = [a_ref[pl.ds(i, 1), :] for i in range(_K)]
    for i in range(_K):
        v = b_ref[pl.ds(i, 1), :]
        for q in range(i, _K):
            fq = fin[q]
            fin[q] = jnp.maximum(fq, v)
            v = jnp.minimum(fq, v)
    for i in range(_K):
        o_ref[pl.ds(i, 1), :] = fin[i]


def _tc_call(n):
    topk = pl.pallas_call(
        _tc_topk_body,
        out_shape=jax.ShapeDtypeStruct((_K, _R), jnp.float32),
    )
    merge = pl.pallas_call(
        _tc_merge_body,
        out_shape=jax.ShapeDtypeStruct((_K, _R), jnp.float32),
    )

    def run(birth, death):
        bt = birth.reshape(8, -1)
        dt = death.reshape(8, -1)
        return topk(bt, dt), merge

    return run


def kernel(pairs):
    n = ((pairs.shape[0] + _PAD - 1) // _PAD) * _PAD
    npad = n - pairs.shape[0]
    birth = jnp.pad(pairs[:, 0], (0, npad), constant_values=jnp.inf)
    death = jnp.pad(pairs[:, 1], (0, npad), constant_values=-jnp.inf)
    out_sc = _sc_call(n)(birth, death)
    out_tc, merge = _tc_call(n)(birth, death)
    return merge(out_tc, out_sc)


# speedup vs baseline: 210.0863x; 1.0611x over previous
import functools

import jax
import jax.numpy as jnp
from jax import lax
from jax.experimental import pallas as pl
from jax.experimental.pallas import tpu as pltpu
from jax.experimental.pallas import tpu_sc as plsc

_K = 5
_R = 1024
_INV_STEP = 1.0 / (_R - 1)

_NW = 32
_CPW = _R // _NW
_L = 16
_NVPW = _CPW // _L

_NB_TC = _R // 128
_PAD = 1024
_SC_TILES = 3

_NET = (
    (0, 1, False), (2, 3, False), (4, 5, False), (6, 7, False),
    (0, 2, False), (1, 3, False), (4, 6, False), (5, 7, False),
    (1, 2, False), (5, 6, False),
    (0, 4, False), (1, 5, False), (2, 6, True), (3, 7, True),
    (2, 4, False), (3, 5, True),
    (1, 2, False), (3, 4, False),
)


@functools.lru_cache(maxsize=None)
def _sc_call(n):
    mesh = plsc.VectorSubcoreMesh(core_axis_name="c", subcore_axis_name="s")
    n_sc = _SC_TILES * _PAD

    @functools.partial(
        pl.kernel,
        mesh=mesh,
        out_type=jax.ShapeDtypeStruct((_K, _R), jnp.float32),
        scratch_types=[
            pltpu.VMEM((n,), jnp.float32),
            pltpu.VMEM((n,), jnp.float32),
            pltpu.VMEM((_K, _CPW), jnp.float32),
        ],
    )
    def body(birth_hbm, death_hbm, out_hbm, b_v, d_v, o_v):
        wid = lax.axis_index("s") * 2 + lax.axis_index("c")
        pltpu.sync_copy(birth_hbm, b_v)
        pltpu.sync_copy(death_hbm, d_v)

        def red(i, carry):
            mn, mx = carry
            return (jnp.minimum(mn, b_v[pl.ds(i * _L, _L)]),
                    jnp.maximum(mx, d_v[pl.ds(i * _L, _L)]))

        mn0 = jnp.full((_L,), jnp.inf, jnp.float32)
        mx0 = jnp.full((_L,), -jnp.inf, jnp.float32)
        mn, mx = lax.fori_loop(0, n // _L, red, (mn0, mx0))
        minb = mn[0]
        maxd = mx[0]
        for i in range(1, _L):
            minb = jnp.minimum(minb, mn[i])
            maxd = jnp.maximum(maxd, mx[i])
        step = (maxd - minb) * jnp.float32(_INV_STEP)

        lane = lax.iota(jnp.int32, _L).astype(jnp.float32)
        base = (wid * _CPW).astype(jnp.float32)
        ts = tuple(minb + (base + jnp.float32(h * _L) + lane) * step
                   for h in range(_NVPW))

        init = (jnp.zeros((_L,), jnp.float32),) * (_K * _NVPW)

        def chunk_body(c, m):
            bv = b_v[pl.ds(c * _L, _L)]
            dv = d_v[pl.ds(c * _L, _L)]
            m = list(m)
            for p in range(_L):
                b = bv[p]
                d = dv[p]
                v = [jnp.minimum(t - b, d - t) for t in ts]
                for i in range(_K):
                    for h in range(_NVPW):
                        mi = m[i * _NVPW + h]
                        m[i * _NVPW + h] = jnp.maximum(mi, v[h])
                        v[h] = jnp.minimum(mi, v[h])
            return tuple(m)

        per_sub = n // 8 // _L
        sc_sub = n_sc // 8 // _L
        m = init
        for s in range(8):
            m = lax.fori_loop(s * per_sub, s * per_sub + sc_sub,
                              chunk_body, m)

        for i in range(_K):
            for h in range(_NVPW):
                o_v[i, pl.ds(h * _L, _L)] = m[i * _NVPW + h]
        for i in range(_K):
            pltpu.sync_copy(o_v.at[i],
                            out_hbm.at[i, pl.ds(wid * _CPW, _CPW)])

    return body


def _tc_topk_body(b_ref, d_ref, o_ref):
    minb = jnp.min(b_ref[...])
    maxd = jnp.max(d_ref[...])
    step = (maxd - minb) * jnp.float32(_INV_STEP)
    lanef = lax.broadcasted_iota(jnp.int32, (8, 128), 1).astype(jnp.float32)
    ts = [minb + (jnp.float32(blk * 128) + lanef) * step
          for blk in range(_NB_TC)]

    ntile = b_ref.shape[1] // 128

    def make_tile_body(blks):
        def tile_body(g, m):
            off = pl.multiple_of(g * 128, 128)
            bt = b_ref[:, pl.ds(off, 128)]
            dt = d_ref[:, pl.ds(off, 128)]
            m = list(m)
            for ub in range(16):
                b8s = [lax.slice(bt, (0, ub * 8 + j), (8, ub * 8 + j + 1))
                       for j in range(8)]
                d8s = [lax.slice(dt, (0, ub * 8 + j), (8, ub * 8 + j + 1))
                       for j in range(8)]
                for bi, blk in enumerate(blks):
                    t = ts[blk]
                    vs = [jnp.minimum(t - b8s[j], d8s[j] - t)
                          for j in range(8)]
                    for a, b, maxonly in _NET:
                        hi = jnp.maximum(vs[a], vs[b])
                        if not maxonly:
                            vs[b] = jnp.minimum(vs[a], vs[b])
                        vs[a] = hi
                    for j in range(_K):
                        v = vs[j]
                        for q in range(j, _K):
                            mi = m[bi * _K + q]
                            m[bi * _K + q] = jnp.maximum(mi, v)
                            v = jnp.minimum(mi, v)
            return tuple(m)

        return tile_body

    t0 = _SC_TILES * _PAD // 1024
    init4 = (jnp.zeros((8, 128), jnp.float32),) * (4 * _K)
    m_lo = lax.fori_loop(t0, ntile, make_tile_body((0, 1, 2, 3)), init4)
    m_hi = lax.fori_loop(t0, ntile, make_tile_body((4, 5, 6, 7)), init4)
    m = list(m_lo) + list(m_hi)

    for blk in range(_NB_TC):
        fin = [jnp.zeros((1, 128), jnp.float32) for _ in range(_K)]
        for s in range(8):
            for i in range(_K):
                v = lax.slice(m[blk * _K + i], (s, 0), (s + 1, 128))
                for q in range(i, _K):
                    fq = fin[q]
                    fin[q] = jnp.maximum(fq, v)
                    v = jnp.minimum(fq, v)
        for i in range(_K):
            o_ref[pl.ds(i, 1), pl.ds(blk * 128, 128)] = fin[i]


def _tc_merge_body(a_ref, b_ref, o_ref):
    fin = [a_ref[pl.ds(i, 1), :] for i in range(_K)]
    for i in range(_K):
        v = b_ref[pl.ds(i, 1), :]
        for q in range(i, _K):
            fq = fin[q]
            fin[q] = jnp.maximum(fq, v)
            v = jnp.minimum(fq, v)
    for i in range(_K):
        o_ref[pl.ds(i, 1), :] = fin[i]


def _tc_call(n):
    topk = pl.pallas_call(
        _tc_topk_body,
        out_shape=jax.ShapeDtypeStruct((_K, _R), jnp.float32),
    )
    merge = pl.pallas_call(
        _tc_merge_body,
        out_shape=jax.ShapeDtypeStruct((_K, _R), jnp.float32),
    )

    def run(birth, death):
        bt = birth.reshape(8, -1)
        dt = death.reshape(8, -1)
        return topk(bt, dt), merge

    return run


def kernel(pairs):
    n = ((pairs.shape[0] + _PAD - 1) // _PAD) * _PAD
    npad = n - pairs.shape[0]
    birth = jnp.pad(pairs[:, 0], (0, npad), constant_values=jnp.inf)
    death = jnp.pad(pairs[:, 1], (0, npad), constant_values=-jnp.inf)
    out_sc = _sc_call(n)(birth, death)
    out_tc, merge = _tc_call(n)(birth, death)
    return merge(out_tc, out_sc)
